# baseline (speedup 1.0000x reference)
import jax
import jax.numpy as jnp
from jax import lax
from jax.experimental import pallas as pl
from jax.experimental.pallas import tpu as pltpu
from jax.experimental.pallas import tpu_sc as plsc

ROWS = 128
N = 32768
LANES = 16
NV = N // LANES
NC = 2
NS = 16
ROWS_PER_WORKER = ROWS // (NC * NS)
S = 4
QV = NV // S
QSHIFT = 13
NB = 2048
HV = NB // LANES


def _to_monotone(x):
  m = jnp.int32(-2147483648)
  s = lax.shift_right_arithmetic(x, 31)
  return lax.bitwise_xor(x, lax.bitwise_or(s, m))


def _from_monotone(u):
  m = jnp.int32(-2147483648)
  s = lax.shift_right_arithmetic(u, 31)
  return lax.bitwise_xor(u, lax.bitwise_or(lax.bitwise_not(s), m))


def _zero16():
  return jnp.zeros((LANES,), jnp.int32)


def _digit(u, shift, mask):
  d = u if shift == 0 else lax.shift_right_logical(u, shift)
  return lax.bitwise_and(d, jnp.int32(mask))


def _sort_body(in_hbm, out_hbm, k0, k1, hist, o0, o1, o2, o3):
  wid = lax.axis_index("s") * NC + lax.axis_index("c")
  offs = [o0, o1, o2, o3]

  def zero_hist(i, c):
    for s in range(S):
      hist[pl.ds((s * NB) + i * LANES, LANES)] = _zero16()
    return c

  def scan_hist(i, carry):
    sl = pl.ds(i * LANES, LANES)
    h = [hist[pl.ds(s * NB + i * LANES, LANES)] for s in range(S)]
    t = h[0]
    for s in range(1, S):
      t = t + h[s]
    inc = plsc.cumsum(t)
    base = inc - t + carry
    for s in range(S):
      offs[s][sl] = base
      hist[pl.ds(s * NB + i * LANES, LANES)] = _zero16()
      if s + 1 < S:
        base = base + h[s]
    return carry + jnp.sum(t)

  def hist_update(key):
    c, last = plsc.scan_count(key)
    plsc.addupdate_scatter(hist, [key], c, mask=last)

  def map_and_hist0(i, c):
    for s in range(S):
      sl = pl.ds((s * QV + i) * LANES, LANES)
      u = _to_monotone(k0[sl])
      k0[sl] = u
      d = _digit(u, 0, 0x7FF)
      hist_update(d + jnp.int32(s * NB))
    return c

  def make_perm(src, dst, shift, mask, next_shift, next_mask, finalize):
    def perm(i, c):
      for s in range(S):
        sl = pl.ds((s * QV + i) * LANES, LANES)
        u = src[sl]
        d = _digit(u, shift, mask)
        cnt, last = plsc.scan_count(d)
        base = plsc.load_gather(offs[s], [d])
        pos = base + cnt - 1
        v = _from_monotone(u) if finalize else u
        plsc.store_scatter(dst, [pos], v)
        plsc.addupdate_scatter(offs[s], [d], cnt, mask=last)
        if next_shift is not None:
          d2 = _digit(u, next_shift, next_mask)
          stripe = lax.shift_right_logical(pos, QSHIFT)
          hist_update(d2 + lax.shift_left(stripe, 11))
      return c
    return perm

  for r in range(ROWS_PER_WORKER):
    row = wid * ROWS_PER_WORKER + r
    pltpu.sync_copy(in_hbm.at[row], k0)
    lax.fori_loop(0, HV, zero_hist, jnp.int32(0))
    lax.fori_loop(0, QV, map_and_hist0, jnp.int32(0))
    lax.fori_loop(0, HV, scan_hist, jnp.int32(0))
    lax.fori_loop(0, QV, make_perm(k0, k1, 0, 0x7FF, 11, 0x7FF, False),
                  jnp.int32(0))
    lax.fori_loop(0, HV, scan_hist, jnp.int32(0))
    lax.fori_loop(0, QV, make_perm(k1, k0, 11, 0x7FF, 22, 0x3FF, False),
                  jnp.int32(0))
    lax.fori_loop(0, HV, scan_hist, jnp.int32(0))
    lax.fori_loop(0, QV, make_perm(k0, k1, 22, 0x3FF, None, None, True),
                  jnp.int32(0))
    pltpu.sync_copy(k1, out_hbm.at[row])


@jax.jit
def kernel(inputs):
  xi = lax.bitcast_convert_type(inputs, jnp.int32)
  mesh = plsc.VectorSubcoreMesh(
      core_axis_name="c", subcore_axis_name="s", num_cores=NC,
      num_subcores=NS)
  sorted_i = pl.kernel(
      _sort_body,
      out_type=jax.ShapeDtypeStruct((ROWS, N), jnp.int32),
      mesh=mesh,
      scratch_types=[
          pltpu.VMEM((N,), jnp.int32),
          pltpu.VMEM((N,), jnp.int32),
          pltpu.VMEM((S * NB,), jnp.int32),
          pltpu.VMEM((NB,), jnp.int32),
          pltpu.VMEM((NB,), jnp.int32),
          pltpu.VMEM((NB,), jnp.int32),
          pltpu.VMEM((NB,), jnp.int32),
      ],
      compiler_params=pltpu.CompilerParams(needs_layout_passes=False),
  )(xi)
  return lax.bitcast_convert_type(sorted_i, jnp.float32)


# speedup vs baseline: 2.1477x; 2.1477x over previous
import jax
import jax.numpy as jnp
from jax import lax
from jax.experimental import pallas as pl
from jax.experimental.pallas import tpu as pltpu
from jax.experimental.pallas import tpu_sc as plsc

ROWS = 128
N = 32768
LANES = 16
NV = N // LANES
NC = 2
NS = 16
ROWS_PER_WORKER = ROWS // (NC * NS)
S = 4
QV = NV // S
NB = 2048
HV = NB // LANES


def _to_monotone(x):
  m = jnp.int32(-2147483648)
  s = lax.shift_right_arithmetic(x, 31)
  return lax.bitwise_xor(x, lax.bitwise_or(s, m))


def _from_monotone(u):
  m = jnp.int32(-2147483648)
  s = lax.shift_right_arithmetic(u, 31)
  return lax.bitwise_xor(u, lax.bitwise_or(lax.bitwise_not(s), m))


def _zero16():
  return jnp.zeros((LANES,), jnp.int32)


def _digit(u, shift, mask):
  d = u if shift == 0 else lax.shift_right_logical(u, shift)
  return lax.bitwise_and(d, jnp.int32(mask))


def _sort_body(in_hbm, out_hbm, k0, k1, h0, h1, h2, h3, o0, o1, o2, o3):
  wid = lax.axis_index("s") * NC + lax.axis_index("c")
  hist = [h0, h1, h2, h3]
  offs = [o0, o1, o2, o3]

  def zero_hist(i, c):
    sl = pl.ds(i * LANES, LANES)
    for s in range(S):
      hist[s][sl] = _zero16()
    return c

  def scan_hist(i, carry):
    sl = pl.ds(i * LANES, LANES)
    h = [hist[s][sl] for s in range(S)]
    t = h[0]
    for s in range(1, S):
      t = t + h[s]
    inc = plsc.cumsum(t)
    base = inc - t + carry
    for s in range(S):
      offs[s][sl] = base
      hist[s][sl] = _zero16()
      if s + 1 < S:
        base = base + h[s]
    return carry + jnp.sum(t)

  def hist_update(s, d):
    c, last = plsc.scan_count(d)
    plsc.addupdate_scatter(hist[s], [d], c, mask=last)

  def map_and_hist0(i):
    for s in range(S):
      sl = pl.ds((s * QV + i) * LANES, LANES)
      u = _to_monotone(k0[sl])
      k0[sl] = u
      hist_update(s, _digit(u, 0, 0x7FF))

  def make_hist(src, shift, mask):
    def histp(i):
      for s in range(S):
        sl = pl.ds((s * QV + i) * LANES, LANES)
        hist_update(s, _digit(src[sl], shift, mask))
    return histp

  def make_perm(src, dst, shift, mask, finalize):
    def perm(i, c):
      for s in range(S):
        sl = pl.ds((s * QV + i) * LANES, LANES)
        u = src[sl]
        d = _digit(u, shift, mask)
        cnt, last = plsc.scan_count(d)
        base = plsc.load_gather(offs[s], [d])
        pos = base + cnt - 1
        v = _from_monotone(u) if finalize else u
        plsc.store_scatter(dst, [pos], v)
        plsc.addupdate_scatter(offs[s], [d], cnt, mask=last)
      return c
    return perm

  for r in range(ROWS_PER_WORKER):
    row = wid * ROWS_PER_WORKER + r
    pltpu.sync_copy(in_hbm.at[row], k0)
    lax.fori_loop(0, HV, zero_hist, jnp.int32(0))
    plsc.parallel_loop(0, QV, step=1, unroll=4)(map_and_hist0)
    lax.fori_loop(0, HV, scan_hist, jnp.int32(0))
    lax.fori_loop(0, QV, make_perm(k0, k1, 0, 0x7FF, False), jnp.int32(0))
    plsc.parallel_loop(0, QV, step=1, unroll=4)(make_hist(k1, 11, 0x7FF))
    lax.fori_loop(0, HV, scan_hist, jnp.int32(0))
    lax.fori_loop(0, QV, make_perm(k1, k0, 11, 0x7FF, False), jnp.int32(0))
    plsc.parallel_loop(0, QV, step=1, unroll=4)(make_hist(k0, 22, 0x3FF))
    lax.fori_loop(0, HV, scan_hist, jnp.int32(0))
    lax.fori_loop(0, QV, make_perm(k0, k1, 22, 0x3FF, True), jnp.int32(0))
    pltpu.sync_copy(k1, out_hbm.at[row])


@jax.jit
def kernel(inputs):
  xi = lax.bitcast_convert_type(inputs, jnp.int32)
  mesh = plsc.VectorSubcoreMesh(
      core_axis_name="c", subcore_axis_name="s", num_cores=NC,
      num_subcores=NS)
  sorted_i = pl.kernel(
      _sort_body,
      out_type=jax.ShapeDtypeStruct((ROWS, N), jnp.int32),
      mesh=mesh,
      scratch_types=[
          pltpu.VMEM((N,), jnp.int32),
          pltpu.VMEM((N,), jnp.int32),
      ] + [pltpu.VMEM((NB,), jnp.int32)] * 8,
      compiler_params=pltpu.CompilerParams(needs_layout_passes=False),
  )(xi)
  return lax.bitcast_convert_type(sorted_i, jnp.float32)


# speedup vs baseline: 3.4553x; 1.6088x over previous
import jax
import jax.numpy as jnp
from jax import lax
from jax.experimental import pallas as pl
from jax.experimental.pallas import tpu as pltpu
from jax.experimental.pallas import tpu_sc as plsc

ROWS = 128
N = 32768
LANES = 16
NV = N // LANES
NC = 2
NS = 16
ROWS_PER_WORKER = ROWS // (NC * NS)
RADIX = 256
VB = 16
BLOCKS = NV // VB
RV = RADIX // LANES


def _to_monotone(x):
  m = jnp.int32(-2147483648)
  s = lax.shift_right_arithmetic(x, 31)
  return lax.bitwise_xor(x, lax.bitwise_or(s, m))


def _from_monotone(u):
  m = jnp.int32(-2147483648)
  s = lax.shift_right_arithmetic(u, 31)
  return lax.bitwise_xor(u, lax.bitwise_or(lax.bitwise_not(s), m))


def _zero16():
  return jnp.zeros((LANES,), jnp.int32)


def _digit(u, shift):
  d = u if shift == 0 else lax.shift_right_logical(u, shift)
  return lax.bitwise_and(d, jnp.int32(RADIX - 1))


def _sort_body(in_hbm, out_hbm, k0, k1, blk, tot):
  wid = lax.axis_index("s") * NC + lax.axis_index("c")

  def zero_blk(i):
    blk[pl.ds(i * LANES, LANES)] = _zero16()

  def make_hist(src, shift, mapped):
    def hist(i):
      sl = pl.ds(i * LANES, LANES)
      u = src[sl]
      if not mapped:
        u = _to_monotone(u)
        src[sl] = u
      d = _digit(u, shift)
      c, last = plsc.scan_count(d)
      base = lax.shift_left(lax.shift_right_logical(i, 4), 8)
      plsc.addupdate_scatter(blk, [d + base], c, mask=last)
    return hist

  def rel_sweep(b, run):
    new_run = []
    for j in range(RV):
      sl = pl.ds(b * RADIX + j * LANES, LANES)
      t = blk[sl]
      blk[sl] = run[j]
      new_run.append(run[j] + t)
    return tuple(new_run)

  def excl_scan(j, carry):
    sl = pl.ds(j * LANES, LANES)
    t = tot[sl]
    inc = plsc.cumsum(t)
    tot[sl] = inc - t + carry
    return carry + jnp.sum(t)

  def base_sweep(i):
    sl = pl.ds(i * LANES, LANES)
    j = lax.rem(i, jnp.int32(RV))
    e = tot[pl.ds(j * LANES, LANES)]
    blk[sl] = blk[sl] + e

  def make_perm(src, dst, shift, finalize):
    def perm(b):
      cbase = b * RADIX
      for v in range(VB):
        sl = pl.ds((b * VB + v) * LANES, LANES)
        u = src[sl]
        d = _digit(u, shift)
        cnt, last = plsc.scan_count(d)
        base = plsc.load_gather(blk, [d + cbase])
        pos = base + cnt - 1
        out = _from_monotone(u) if finalize else u
        plsc.store_scatter(dst, [pos], out)
        plsc.addupdate_scatter(blk, [d + cbase], cnt, mask=last)
      for j in range(RV):
        blk[pl.ds(b * RADIX + j * LANES, LANES)] = _zero16()
    return perm

  def do_pass(src, dst, shift, mapped, finalize):
    plsc.parallel_loop(0, NV, step=1, unroll=4)(make_hist(src, shift, mapped))
    run = lax.fori_loop(0, BLOCKS, rel_sweep,
                        tuple(_zero16() for _ in range(RV)))
    for j in range(RV):
      tot[pl.ds(j * LANES, LANES)] = run[j]
    lax.fori_loop(0, RV, excl_scan, jnp.int32(0))
    plsc.parallel_loop(0, NV, step=1, unroll=4)(base_sweep)
    plsc.parallel_loop(0, BLOCKS, step=1)(make_perm(src, dst, shift, finalize))

  def row_body(r, c):
    row = wid * ROWS_PER_WORKER + r
    pltpu.sync_copy(in_hbm.at[row], k0)
    do_pass(k0, k1, 0, False, False)
    do_pass(k1, k0, 8, True, False)
    do_pass(k0, k1, 16, True, False)
    do_pass(k1, k0, 24, True, True)
    pltpu.sync_copy(k0, out_hbm.at[row])
    return c

  plsc.parallel_loop(0, NV, step=1, unroll=4)(zero_blk)
  lax.fori_loop(0, ROWS_PER_WORKER, row_body, jnp.int32(0))


@jax.jit
def kernel(inputs):
  xi = lax.bitcast_convert_type(inputs, jnp.int32)
  mesh = plsc.VectorSubcoreMesh(
      core_axis_name="c", subcore_axis_name="s", num_cores=NC,
      num_subcores=NS)
  sorted_i = pl.kernel(
      _sort_body,
      out_type=jax.ShapeDtypeStruct((ROWS, N), jnp.int32),
      mesh=mesh,
      scratch_types=[
          pltpu.VMEM((N,), jnp.int32),
          pltpu.VMEM((N,), jnp.int32),
          pltpu.VMEM((BLOCKS * RADIX,), jnp.int32),
          pltpu.VMEM((RADIX,), jnp.int32),
      ],
      compiler_params=pltpu.CompilerParams(needs_layout_passes=False),
  )(xi)
  return lax.bitcast_convert_type(sorted_i, jnp.float32)
